# baseline (speedup 1.0000x reference)
import jax
import jax.numpy as jnp
from jax import lax
from jax.experimental import pallas as pl
from jax.experimental.pallas import tpu as pltpu

TOKENS = 32768
HIDDEN = 4096
N_EXPERTS = 64
BLOCK_M = 512
NBLK = TOKENS // BLOCK_M


def _outer(x_hbm, w_ref, out_hbm):
    w16 = w_ref[...].astype(jnp.bfloat16)

    def inner(x_blk, out_blk):
        x16 = x_blk[...].astype(jnp.bfloat16)
        out_blk[...] = lax.dot_general(
            x16, w16, (((1,), (1,)), ((), ())),
            preferred_element_type=jnp.float32)

    pltpu.emit_pipeline(
        inner,
        grid=(NBLK,),
        in_specs=[
            pl.BlockSpec((BLOCK_M, HIDDEN), lambda i: (i, 0),
                         pipeline_mode=pl.Buffered(buffer_count=4)),
        ],
        out_specs=[
            pl.BlockSpec((BLOCK_M, N_EXPERTS), lambda i: (i, 0)),
        ],
    )(x_hbm, out_hbm)


def kernel(hidden_states, W):
    return pl.pallas_call(
        _outer,
        in_specs=[
            pl.BlockSpec(memory_space=pltpu.MemorySpace.HBM),
            pl.BlockSpec(memory_space=pltpu.MemorySpace.VMEM),
        ],
        out_specs=pl.BlockSpec(memory_space=pltpu.MemorySpace.HBM),
        out_shape=jax.ShapeDtypeStruct((TOKENS, N_EXPERTS), jnp.float32),
    )(hidden_states, W)


# speedup vs baseline: 1.0190x; 1.0190x over previous
import jax
import jax.numpy as jnp
from jax.experimental import pallas as pl
from jax.experimental.pallas import tpu as pltpu

TOKENS = 32768
HIDDEN = 4096
N_EXPERTS = 64
BLOCK_M = 512


def _router_kernel(x_ref, w_ref, out_ref, w16_ref):
    @pl.when(pl.program_id(0) == 0)
    def _cast_w():
        w16_ref[...] = w_ref[...].astype(jnp.bfloat16)

    x16 = x_ref[...].astype(jnp.bfloat16)
    out_ref[...] = jax.lax.dot_general(
        x16, w16_ref[...], (((1,), (1,)), ((), ())),
        preferred_element_type=jnp.float32)


def kernel(hidden_states, W):
    grid = (TOKENS // BLOCK_M,)
    return pl.pallas_call(
        _router_kernel,
        grid=grid,
        in_specs=[
            pl.BlockSpec((BLOCK_M, HIDDEN), lambda i: (i, 0)),
            pl.BlockSpec((N_EXPERTS, HIDDEN), lambda i: (0, 0)),
        ],
        out_specs=pl.BlockSpec((BLOCK_M, N_EXPERTS), lambda i: (i, 0)),
        out_shape=jax.ShapeDtypeStruct((TOKENS, N_EXPERTS), jnp.float32),
        scratch_shapes=[
            pltpu.VMEM((N_EXPERTS, HIDDEN), jnp.bfloat16),
        ],
        compiler_params=pltpu.CompilerParams(
            dimension_semantics=("arbitrary",),
        ),
    )(hidden_states, W)
